# baseline (speedup 1.0000x reference)
import jax
import jax.numpy as jnp
from jax.experimental import pallas as pl
from jax.experimental.pallas import tpu as pltpu


def _copy_body(u_in, i_in, u_out, i_out, sem_u, sem_i):
    cu = pltpu.make_async_copy(u_in, u_out, sem_u)
    ci = pltpu.make_async_copy(i_in, i_out, sem_i)
    cu.start()
    ci.start()
    cu.wait()
    ci.wait()


def kernel(user_emb, item_emb):
    ut = user_emb.T
    it = item_emb.T
    out_ut, out_it = pl.pallas_call(
        _copy_body,
        out_shape=(
            jax.ShapeDtypeStruct(ut.shape, ut.dtype),
            jax.ShapeDtypeStruct(it.shape, it.dtype),
        ),
        in_specs=[
            pl.BlockSpec(memory_space=pl.ANY),
            pl.BlockSpec(memory_space=pl.ANY),
        ],
        out_specs=[
            pl.BlockSpec(memory_space=pl.ANY),
            pl.BlockSpec(memory_space=pl.ANY),
        ],
        scratch_shapes=[
            pltpu.SemaphoreType.DMA,
            pltpu.SemaphoreType.DMA,
        ],
    )(ut, it)
    return out_ut.T, out_it.T


# speedup vs baseline: 35.2542x; 35.2542x over previous
import functools

import jax
import jax.numpy as jnp
from jax import lax
from jax.experimental import pallas as pl
from jax.experimental.pallas import tpu as pltpu
from jax.experimental.pallas import tpu_sc as plsc

CH = 2048


def kernel(user_emb, item_emb):
    ut = user_emb.T
    it = item_emb.T
    d, n = ut.shape
    info = plsc.get_sparse_core_info()
    nw = info.num_cores * info.num_subcores
    nc = info.num_cores
    nfull = n // CH
    rem = n - nfull * CH
    cpw = pl.cdiv(nfull, nw)

    scratch = [pltpu.VMEM((d, CH), jnp.float32)]
    if rem:
        scratch.append(pltpu.VMEM((d, rem), jnp.float32))

    mesh = plsc.VectorSubcoreMesh(core_axis_name="c", subcore_axis_name="s")

    @functools.partial(
        pl.kernel,
        mesh=mesh,
        out_type=(
            jax.ShapeDtypeStruct(ut.shape, ut.dtype),
            jax.ShapeDtypeStruct(it.shape, it.dtype),
        ),
        scratch_types=scratch,
    )
    def sc_copy(u_in, i_in, u_out, i_out, buf, *rembuf):
        wid = lax.axis_index("s") * nc + lax.axis_index("c")

        def copy_table(src, dst):
            for k in range(cpw):
                c = wid + k * nw

                @pl.when(c < nfull)
                def _():
                    s = pl.ds(c * CH, CH)
                    pltpu.sync_copy(src.at[:, s], buf)
                    pltpu.sync_copy(buf, dst.at[:, s])

        copy_table(u_in, u_out)
        copy_table(i_in, i_out)

        if rem:

            @pl.when(wid == nw - 1)
            def _():
                s = pl.ds(nfull * CH, rem)
                pltpu.sync_copy(u_in.at[:, s], rembuf[0])
                pltpu.sync_copy(rembuf[0], u_out.at[:, s])
                pltpu.sync_copy(i_in.at[:, s], rembuf[0])
                pltpu.sync_copy(rembuf[0], i_out.at[:, s])

    out_ut, out_it = sc_copy(ut, it)
    return out_ut.T, out_it.T


# speedup vs baseline: 40.6441x; 1.1529x over previous
import functools

import jax
import jax.numpy as jnp
from jax import lax
from jax.experimental import pallas as pl
from jax.experimental.pallas import tpu as pltpu
from jax.experimental.pallas import tpu_sc as plsc

TC_BLOCK = 32768
CH = 2048


def _tc_copy_body(src, dst):
    dst[...] = src[...]


def _tc_copy(x):
    d, n = x.shape
    spec = pl.BlockSpec((d, TC_BLOCK), lambda g: (0, g))
    return pl.pallas_call(
        _tc_copy_body,
        grid=(pl.cdiv(n, TC_BLOCK),),
        out_shape=jax.ShapeDtypeStruct(x.shape, x.dtype),
        in_specs=[spec],
        out_specs=spec,
    )(x)


def _sc_copy(x):
    d, n = x.shape
    info = plsc.get_sparse_core_info()
    nw = info.num_cores * info.num_subcores
    nc = info.num_cores
    nfull = n // CH
    rem = n - nfull * CH
    cpw = pl.cdiv(nfull, nw)

    scratch = [pltpu.VMEM((d, CH), jnp.float32)]
    if rem:
        scratch.append(pltpu.VMEM((d, rem), jnp.float32))

    mesh = plsc.VectorSubcoreMesh(core_axis_name="c", subcore_axis_name="s")

    @functools.partial(
        pl.kernel,
        mesh=mesh,
        out_type=jax.ShapeDtypeStruct(x.shape, x.dtype),
        scratch_types=scratch,
    )
    def sc_copy(src, dst, buf, *rembuf):
        wid = lax.axis_index("s") * nc + lax.axis_index("c")
        for k in range(cpw):
            c = wid + k * nw

            @pl.when(c < nfull)
            def _():
                s = pl.ds(c * CH, CH)
                pltpu.sync_copy(src.at[:, s], buf)
                pltpu.sync_copy(buf, dst.at[:, s])

        if rem:

            @pl.when(wid == nw - 1)
            def _():
                s = pl.ds(nfull * CH, rem)
                pltpu.sync_copy(src.at[:, s], rembuf[0])
                pltpu.sync_copy(rembuf[0], dst.at[:, s])

    return sc_copy(x)


def kernel(user_emb, item_emb):
    ut = user_emb.T
    it = item_emb.T
    out_ut = _tc_copy(ut)
    out_it = _sc_copy(it)
    return out_ut.T, out_it.T
